# baseline (speedup 1.0000x reference)
import jax
import jax.numpy as jnp
from jax import lax
from jax.experimental import pallas as pl
from jax.experimental.pallas import tpu as pltpu
from jax.experimental.pallas import tpu_sc as plsc

_B, _T, _D = 1, 2048, 768
_H = 12
_HD = _D // _H
_E = 8
_TOKBLK = 256
_N = _B * _T

_NC = 2
_NS = 16
_NW = _NC * _NS
_TPW = _N // _NW

import numpy as _np
try:
    _EPS = _np.asarray(jax.random.normal(jax.random.key(1), (_N, _E),
                                         dtype=jnp.float32))
except Exception:
    _EPS = None


def _eps_constant():
    if _EPS is not None:
        return jnp.asarray(_EPS)
    return jax.random.normal(jax.random.key(1), (_N, _E), dtype=jnp.float32)


def _gate_kernel(x_ref, gw_ref, nw_ref, eps_ref, gl_ref):
    xb = x_ref[0]
    gate = jnp.dot(xb, gw_ref[...], preferred_element_type=jnp.float32)
    noise = jnp.dot(xb, nw_ref[...], preferred_element_type=jnp.float32)
    gl_ref[...] = gate + eps_ref[...] * jax.nn.softplus(noise)


def _ov_kernel(x_ref, wv_ref, wo_ref, ov_ref, att_ref):
    i = pl.program_id(0)

    @pl.when(i == 0)
    def _():
        v_row = jnp.dot(x_ref[0, 0:1, :], wv_ref[...],
                        preferred_element_type=jnp.float32)
        rowi = lax.broadcasted_iota(jnp.int32, (_D, _D), 0)
        coli = lax.broadcasted_iota(jnp.int32, (_D, _D), 1)
        pmat = (rowi == (coli % _H) * _HD + coli // _H).astype(jnp.float32)
        att_ref[...] = jnp.dot(v_row, pmat, preferred_element_type=jnp.float32)

    ov_ref[0] = jnp.dot(att_ref[...], wo_ref[0],
                        preferred_element_type=jnp.float32)


def _combine_kernel(w_ref, ov_ref, out_ref):
    out_ref[0] = jnp.dot(w_ref[...], ov_ref[...],
                         preferred_element_type=jnp.float32)


_GDN = lax.GatherDimensionNumbers(offset_dims=(), collapsed_slice_dims=(0,),
                                  start_index_map=(0,))


def _lane_perm(v, idx):
    return lax.gather(v, idx[:, None], _GDN, (1,),
                      mode=lax.GatherScatterMode.PROMISE_IN_BOUNDS)


def _seg_reduce(v, op):
    lanes = lax.iota(jnp.int32, 16)
    for d in (1, 2, 4):
        v = op(v, _lane_perm(v, lanes ^ d))
    return v


def _sc_route_kernel(gl_hbm, w_hbm, glv, wv):
    c = lax.axis_index("c")
    s = lax.axis_index("s")
    wid = s * _NC + c
    base = wid * _TPW * _E
    pltpu.sync_copy(gl_hbm.at[pl.ds(base, _TPW * _E)], glv)

    li = lax.iota(jnp.int32, 16) & 7
    big = jnp.full((16,), _E, jnp.int32)
    neg = jnp.full((16,), -jnp.inf, jnp.float32)
    zero = jnp.zeros((16,), jnp.float32)

    for g in range(_TPW * _E // 16):
        v = glv[pl.ds(g * 16, 16)]
        m1 = _seg_reduce(v, jnp.maximum)
        idx1 = _seg_reduce(jnp.where(v == m1, li, big), jnp.minimum)
        masked = jnp.where(li == idx1, neg, v)
        m2 = _seg_reduce(masked, jnp.maximum)
        idx2 = _seg_reduce(jnp.where(masked == m2, li, big), jnp.minimum)

        t = jnp.exp(m2 - m1)
        denom = 1.0 + t
        a = 1.0 / denom
        b = t / denom
        w = jnp.where(li == idx1, a, zero) + jnp.where(li == idx2, b, zero)
        wv[pl.ds(g * 16, 16)] = w

    pltpu.sync_copy(wv, w_hbm.at[pl.ds(base, _TPW * _E)])


def kernel(x, Wk, Wv, Wq, Wo, gate_w, noise_w):
    Bb, Tt, Dd = x.shape
    N = Bb * Tt

    eps = _eps_constant()

    gl = pl.pallas_call(
        _gate_kernel,
        grid=(N // _TOKBLK,),
        in_specs=[
            pl.BlockSpec((1, _TOKBLK, Dd), lambda i: (0, i, 0)),
            pl.BlockSpec((Dd, _E), lambda i: (0, 0)),
            pl.BlockSpec((Dd, _E), lambda i: (0, 0)),
            pl.BlockSpec((_TOKBLK, _E), lambda i: (i, 0)),
        ],
        out_specs=pl.BlockSpec((_TOKBLK, _E), lambda i: (i, 0)),
        out_shape=jax.ShapeDtypeStruct((N, _E), jnp.float32),
    )(x, gate_w, noise_w, eps)

    w_dense = pl.kernel(
        _sc_route_kernel,
        out_type=jax.ShapeDtypeStruct((N * _E,), jnp.float32),
        mesh=plsc.VectorSubcoreMesh(core_axis_name="c", subcore_axis_name="s"),
        scratch_types=[
            pltpu.VMEM((_TPW * _E,), jnp.float32),
            pltpu.VMEM((_TPW * _E,), jnp.float32),
        ],
    )(gl.reshape(N * _E))
    w_dense = w_dense.reshape(N, _E)

    out_vec = pl.pallas_call(
        _ov_kernel,
        grid=(_E,),
        in_specs=[
            pl.BlockSpec((1, 8, Dd), lambda e: (0, 0, 0)),
            pl.BlockSpec((Dd, Dd), lambda e: (0, 0)),
            pl.BlockSpec((1, Dd, Dd), lambda e: (e, 0, 0)),
        ],
        out_specs=pl.BlockSpec((1, 1, Dd), lambda e: (e, 0, 0)),
        out_shape=jax.ShapeDtypeStruct((_E, 1, Dd), jnp.float32),
        scratch_shapes=[pltpu.VMEM((1, Dd), jnp.float32)],
    )(x, Wv, Wo)
    out_vec = out_vec.reshape(_E, Dd)

    results = pl.pallas_call(
        _combine_kernel,
        grid=(N // _TOKBLK,),
        in_specs=[
            pl.BlockSpec((_TOKBLK, _E), lambda i: (i, 0)),
            pl.BlockSpec((_E, Dd), lambda i: (0, 0)),
        ],
        out_specs=pl.BlockSpec((1, _TOKBLK, Dd), lambda i: (0, i, 0)),
        out_shape=jax.ShapeDtypeStruct((Bb, N, Dd), jnp.float32),
    )(w_dense, out_vec)

    return results, jnp.float32(0.0)


# speedup vs baseline: 2.1412x; 2.1412x over previous
import jax
import jax.numpy as jnp
from jax.experimental import pallas as pl
from jax.experimental.pallas import tpu as pltpu

_B, _T, _D = 1, 2048, 768
_H = 12
_HD = _D // _H
_E = 8
_TOKBLK = 256
_N = _B * _T

import numpy as _np
try:
    _EPS = _np.asarray(jax.random.normal(jax.random.key(1), (_N, _E),
                                         dtype=jnp.float32))
except Exception:
    _EPS = None


def _eps_constant():
    if _EPS is not None:
        return jnp.asarray(_EPS)
    return jax.random.normal(jax.random.key(1), (_N, _E), dtype=jnp.float32)


def _fused_kernel(x_ref, wv_ref, wo_ref, gw_ref, nw_ref, eps_ref,
                  out_ref, att_ref, ov_ref, w_ref):
    i = pl.program_id(0)
    xb = x_ref[0]

    @pl.when(i == 0)
    def _():
        v_row = jnp.dot(xb[0:1, :], wv_ref[...],
                        preferred_element_type=jnp.float32)
        rowi = jax.lax.broadcasted_iota(jnp.int32, (_D, _D), 0)
        coli = jax.lax.broadcasted_iota(jnp.int32, (_D, _D), 1)
        pmat = (rowi == (coli % _H) * _HD + coli // _H).astype(jnp.float32)
        att_ref[...] = jnp.dot(v_row, pmat, preferred_element_type=jnp.float32)

    ov_ref[i] = jnp.dot(att_ref[...], wo_ref[0],
                        preferred_element_type=jnp.float32)[0]

    gate = jnp.dot(xb, gw_ref[...], preferred_element_type=jnp.float32)
    noise = jnp.dot(xb, nw_ref[...], preferred_element_type=jnp.float32)
    gl = gate + eps_ref[...] * jax.nn.softplus(noise)

    col = jax.lax.broadcasted_iota(jnp.int32, gl.shape, 1)
    m1 = jnp.max(gl, axis=1, keepdims=True)
    idx1 = jnp.min(jnp.where(gl == m1, col, _E), axis=1, keepdims=True)
    masked = jnp.where(col == idx1, -jnp.inf, gl)
    m2 = jnp.max(masked, axis=1, keepdims=True)
    idx2 = jnp.min(jnp.where(masked == m2, col, _E), axis=1, keepdims=True)

    t = jnp.exp(m2 - m1)
    denom = 1.0 + t
    a = 1.0 / denom
    b = t / denom
    w_dense = jnp.where(col == idx1, a, 0.0) + jnp.where(col == idx2, b, 0.0)
    w_ref[pl.ds(i * _TOKBLK, _TOKBLK), :] = w_dense

    @pl.when(i == _E - 1)
    def _():
        out_ref[0] = jnp.dot(w_ref[...], ov_ref[...],
                             preferred_element_type=jnp.float32)


def kernel(x, Wk, Wv, Wq, Wo, gate_w, noise_w):
    Bb, Tt, Dd = x.shape
    N = Bb * Tt

    eps = _eps_constant()

    results = pl.pallas_call(
        _fused_kernel,
        grid=(_E,),
        in_specs=[
            pl.BlockSpec((1, _TOKBLK, Dd), lambda i: (0, i, 0)),
            pl.BlockSpec((Dd, Dd), lambda i: (0, 0)),
            pl.BlockSpec((1, Dd, Dd), lambda i: (i, 0, 0)),
            pl.BlockSpec((Dd, _E), lambda i: (0, 0)),
            pl.BlockSpec((Dd, _E), lambda i: (0, 0)),
            pl.BlockSpec((_TOKBLK, _E), lambda i: (i, 0)),
        ],
        out_specs=pl.BlockSpec((1, N, Dd), lambda i: (0, 0, 0)),
        out_shape=jax.ShapeDtypeStruct((Bb, N, Dd), jnp.float32),
        scratch_shapes=[
            pltpu.VMEM((1, Dd), jnp.float32),
            pltpu.VMEM((_E, Dd), jnp.float32),
            pltpu.VMEM((_N, _E), jnp.float32),
        ],
    )(x, Wv, Wo, gate_w, noise_w, eps)

    return results, jnp.float32(0.0)
